# baseline (speedup 1.0000x reference)
import jax
import jax.numpy as jnp
from jax import lax
from jax.experimental import pallas as pl
from jax.experimental.pallas import tpu as pltpu
from jax.experimental.pallas import tpu_sc as plsc

_NUM_EMB = 1000000
_NPAIR = _NUM_EMB // 2
_DIM = 64
_B = 16384
_L = 50

_INFO = plsc.get_sparse_core_info()
_NC = _INFO.num_cores
_NS = _INFO.num_subcores
_NW = _NC * _NS

_BBLK = 128
_NBB = _B // _BBLK
_BB_PW = _NBB // _NW


def _transpose_chunk(lrow, hq, raw, outv):
    iota = lax.iota(jnp.int32, 16)
    for bg in range(8):
        hvec = hq[lrow, pl.ds(bg * 16, 16)]
        h64 = hvec * 64
        rows = iota + (bg * 16)
        for d in range(_DIM):
            outv[d, pl.ds(bg * 16, 16)] = plsc.load_gather(
                raw, [rows, h64 + d])


def _body(idxT_hbm, pairs_hbm, out_hbm,
          idxq, hq, raw0, raw1, outv0, outv1,
          sg0, sg1, sw0, sw1):
    raws = (raw0, raw1)
    outs = (outv0, outv1)
    sg = (sg0, sg1)
    sw = (sw0, sw1)

    wid = lax.axis_index("s") * _NC + lax.axis_index("c")

    def one_bblk(bb, carry0):
        b0 = (wid * _BB_PW + bb) * _BBLK

        for t in range(6):
            pltpu.sync_copy(
                idxT_hbm.at[pl.ds(t * 8, 8), pl.ds(b0, _BBLK)],
                idxq.at[pl.ds(t * 8, 8)])
        pltpu.sync_copy(
            idxT_hbm.at[pl.ds(48, 2), pl.ds(b0, _BBLK)],
            idxq.at[pl.ds(48, 2)])

        def prep(i, carry):
            for c in range(8):
                v = idxq[i, pl.ds(c * 16, 16)]
                hq[i, pl.ds(c * 16, 16)] = lax.bitwise_and(v, 1)
                idxq[i, pl.ds(c * 16, 16)] = lax.shift_right_logical(v, 1)
            return carry

        lax.fori_loop(0, _L, prep, 0)

        pltpu.async_copy(pairs_hbm.at[idxq.at[0]], raw0, sg0)
        pltpu.async_copy(pairs_hbm.at[idxq.at[1]], raw1, sg1)

        def two_l(h, carry):
            for j in range(2):
                l = 2 * h + j
                pltpu.make_async_copy(
                    pairs_hbm.at[idxq.at[l]], raws[j], sg[j]).wait()

                @pl.when(l >= 2)
                def _():
                    pltpu.make_async_copy(
                        outs[j], out_hbm.at[l, :, pl.ds(b0, _BBLK)],
                        sw[j]).wait()

                _transpose_chunk(l, hq, raws[j], outs[j])

                @pl.when(l + 2 < _L)
                def _():
                    pltpu.async_copy(
                        pairs_hbm.at[idxq.at[l + 2]], raws[j], sg[j])

                pltpu.async_copy(
                    outs[j], out_hbm.at[l, :, pl.ds(b0, _BBLK)], sw[j])
            return carry

        lax.fori_loop(0, _L // 2, two_l, 0)

        for j in range(2):
            pltpu.make_async_copy(
                outs[j], out_hbm.at[48 + j, :, pl.ds(b0, _BBLK)],
                sw[j]).wait()
        return carry0

    lax.fori_loop(0, _BB_PW, one_bblk, 0)


_mesh = plsc.VectorSubcoreMesh(core_axis_name="c", subcore_axis_name="s")

_gather = pl.kernel(
    _body,
    out_type=jax.ShapeDtypeStruct((_L, _DIM, _B), jnp.float32),
    mesh=_mesh,
    scratch_types=[
        pltpu.VMEM((56, _BBLK), jnp.int32),
        pltpu.VMEM((56, _BBLK), jnp.int32),
        pltpu.VMEM((_BBLK, 128), jnp.float32),
        pltpu.VMEM((_BBLK, 128), jnp.float32),
        pltpu.VMEM((_DIM, _BBLK), jnp.float32),
        pltpu.VMEM((_DIM, _BBLK), jnp.float32),
        pltpu.SemaphoreType.DMA,
        pltpu.SemaphoreType.DMA,
        pltpu.SemaphoreType.DMA,
        pltpu.SemaphoreType.DMA,
    ],
    compiler_params=pltpu.CompilerParams(
        use_tc_tiling_on_sc=True, needs_layout_passes=False,
        disable_bounds_checks=True),
)


@jax.jit
def kernel(indices, table):
    idxT = indices.T
    pairs = table.reshape(_NPAIR, 2 * _DIM)
    out_phys = _gather(idxT, pairs)
    return jnp.transpose(out_phys, (2, 0, 1))


# speedup vs baseline: 1.5625x; 1.5625x over previous
import jax
import jax.numpy as jnp
from jax import lax
from jax.experimental import pallas as pl
from jax.experimental.pallas import tpu as pltpu
from jax.experimental.pallas import tpu_sc as plsc

_NUM_EMB = 1000000
_DIM = 64
_B = 16384
_L = 50

_INFO = plsc.get_sparse_core_info()
_NC = _INFO.num_cores
_NS = _INFO.num_subcores
_NW = _NC * _NS

_N = _B * _L
_IW = 128
_NROWS = _N // _IW
_ROWS_PW = _NROWS // _NW
_K = 5
_STEPS = _ROWS_PW // _K
_CHUNK = _K * _IW


def _body(idx_hbm, table_hbm, out_hbm,
          idx0, idx1, rows0, rows1,
          sg0, sg1, sw0, sw1, si0, si1):
    idx_bufs = (idx0, idx1)
    rows_bufs = (rows0, rows1)
    sg = (sg0, sg1)
    sw = (sw0, sw1)
    si = (si0, si1)

    wid = lax.axis_index("s") * _NC + lax.axis_index("c")
    row0 = wid * _ROWS_PW

    pltpu.async_copy(idx_hbm.at[pl.ds(row0, _K)], idx0, si0)
    pltpu.async_copy(idx_hbm.at[pl.ds(row0 + _K, _K)], idx1, si1)

    def two_chunks(h, carry):
        for b in range(2):
            g = h * 2 + b
            r0 = row0 + g * _K
            pltpu.make_async_copy(
                idx_hbm.at[pl.ds(row0, _K)], idx_bufs[b], si[b]).wait()

            @pl.when(g >= 2)
            def _():
                pltpu.make_async_copy(
                    rows_bufs[b], out_hbm.at[pl.ds(r0 * _IW, _CHUNK)],
                    sw[b]).wait()

            for j in range(_K):
                pltpu.async_copy(
                    table_hbm.at[idx_bufs[b].at[j]],
                    rows_bufs[b].at[pl.ds(j * _IW, _IW)],
                    sg[b],
                )
            for j in range(_K):
                pltpu.make_async_copy(
                    table_hbm.at[idx_bufs[b].at[j]],
                    rows_bufs[b].at[pl.ds(j * _IW, _IW)],
                    sg[b],
                ).wait()

            @pl.when(g + 2 < _STEPS)
            def _():
                pltpu.async_copy(
                    idx_hbm.at[pl.ds(r0 + 2 * _K, _K)], idx_bufs[b], si[b])

            pltpu.async_copy(
                rows_bufs[b], out_hbm.at[pl.ds(r0 * _IW, _CHUNK)], sw[b])
        return carry

    lax.fori_loop(0, _STEPS // 2, two_chunks, 0)

    for b in range(2):
        pltpu.make_async_copy(
            rows_bufs[b], out_hbm.at[pl.ds(row0 * _IW, _CHUNK)], sw[b]).wait()


_mesh = plsc.VectorSubcoreMesh(core_axis_name="c", subcore_axis_name="s")

_gather = pl.kernel(
    _body,
    out_type=jax.ShapeDtypeStruct((_N, _DIM), jnp.float32),
    mesh=_mesh,
    scratch_types=[
        pltpu.VMEM((_K, _IW), jnp.int32),
        pltpu.VMEM((_K, _IW), jnp.int32),
        pltpu.VMEM((_CHUNK, _DIM), jnp.float32),
        pltpu.VMEM((_CHUNK, _DIM), jnp.float32),
        pltpu.SemaphoreType.DMA,
        pltpu.SemaphoreType.DMA,
        pltpu.SemaphoreType.DMA,
        pltpu.SemaphoreType.DMA,
        pltpu.SemaphoreType.DMA,
        pltpu.SemaphoreType.DMA,
    ],
    compiler_params=pltpu.CompilerParams(
        use_tc_tiling_on_sc=False, disable_bounds_checks=True),
)


@jax.jit
def kernel(indices, table):
    idx2d = indices.reshape(_NROWS, _IW)
    out = _gather(idx2d, table)
    return out.reshape(_B, _L, _DIM)
